# baseline (speedup 1.0000x reference)
import functools

import jax
import jax.numpy as jnp
from jax import lax
from jax.experimental import pallas as pl
from jax.experimental.pallas import tpu as pltpu

N = 10000
E = 320000
D = 128
MID = 256
HID = 256
NCLS = 32
NV = 64
NE = 16
G = 64
EPS = 1e-7

BN = 960
NP_ = 11520
NB = NP_ // BN

_HIGH = lax.Precision.HIGHEST

INTERPRET = False


def _tc1_body(c_ref, x_ref, v_ref, e_ref, w1_ref, b1_ref,
              hmid_ref, stats_ref, exp_s, expm_s, acc_s):
    i = pl.program_id(0)

    @pl.when(i == 0)
    def _init():
        ee = e_ref[...]
        for vi in range(NV):
            m = jax.nn.relu(v_ref[vi:vi + 1, :] + ee) + EPS
            ex = jnp.exp(m)
            exp_s[vi * NE:(vi + 1) * NE, :] = ex
            expm_s[vi * NE:(vi + 1) * NE, :] = ex * m
        acc_s[...] = jnp.zeros_like(acc_s)

    den = jnp.zeros((BN, D), jnp.float32)
    num = jnp.zeros((BN, D), jnp.float32)
    for kh in range(8):
        cb = c_ref[kh].reshape(BN, 128).astype(jnp.float32)
        den = den + jnp.dot(cb, exp_s[kh * 128:(kh + 1) * 128, :])
        num = num + jnp.dot(cb, expm_s[kh * 128:(kh + 1) * 128, :])
    aggr = num / (den + 1e-16)

    xb = x_ref[...]
    oh = (xb == lax.broadcasted_iota(jnp.int32, (1, NV), 1)).astype(jnp.float32)
    h = jnp.dot(oh, v_ref[...], precision=_HIGH)

    out = aggr + h
    hmid = jnp.dot(out, w1_ref[...], precision=_HIGH) + b1_ref[...]
    hmid_ref[...] = hmid

    rid = i * BN + lax.broadcasted_iota(jnp.int32, (BN, 1), 0)
    hm = jnp.where(rid < N, hmid, 0.0)
    acc_s[0:1, :] += jnp.sum(hm, axis=0, keepdims=True)
    acc_s[1:2, :] += jnp.sum(hm * hm, axis=0, keepdims=True)

    @pl.when(i == NB - 1)
    def _fin():
        stats_ref[...] = acc_s[...]


def _tc1(c, x2d, v_emb, e_emb, W1, b1):
    return pl.pallas_call(
        _tc1_body,
        grid=(NB,),
        in_specs=[
            pl.BlockSpec((8, BN // 16, 16, 128), lambda i: (0, i, 0, 0)),
            pl.BlockSpec((BN, 1), lambda i: (i, 0)),
            pl.BlockSpec((NV, D), lambda i: (0, 0)),
            pl.BlockSpec((NE, D), lambda i: (0, 0)),
            pl.BlockSpec((D, MID), lambda i: (0, 0)),
            pl.BlockSpec((1, MID), lambda i: (0, 0)),
        ],
        out_specs=[
            pl.BlockSpec((BN, MID), lambda i: (i, 0)),
            pl.BlockSpec((2, MID), lambda i: (0, 0)),
        ],
        out_shape=[
            jax.ShapeDtypeStruct((NP_, MID), jnp.float32),
            jax.ShapeDtypeStruct((2, MID), jnp.float32),
        ],
        scratch_shapes=[
            pltpu.VMEM((NV * NE, D), jnp.float32),
            pltpu.VMEM((NV * NE, D), jnp.float32),
            pltpu.VMEM((2, MID), jnp.float32),
        ],
        interpret=INTERPRET,
    )(c, x2d, v_emb, e_emb, W1, b1)


def _tc2_body(hmid_ref, stats_ref, batch_ref, bng_ref, bnb_ref,
              w2_ref, b2_ref, fw1_ref, fb1_ref, fw2_ref, fb2_ref,
              res_ref, pool_s, cnt_s):
    i = pl.program_id(0)

    @pl.when(i == 0)
    def _init():
        pool_s[...] = jnp.zeros_like(pool_s)
        cnt_s[...] = jnp.zeros_like(cnt_s)

    st = stats_ref[...]
    mu = st[0:1, :] / N
    var = st[1:2, :] / N - mu * mu
    inv = lax.rsqrt(var + 1e-5)

    hb = hmid_ref[...]
    relu_h = jax.nn.relu((hb - mu) * inv * bng_ref[...] + bnb_ref[...])

    bb = batch_ref[0]
    ohT = (lax.broadcasted_iota(jnp.int32, (G, 1), 0) == bb).astype(jnp.float32)
    pool_s[...] += jnp.dot(ohT, relu_h, precision=_HIGH)
    cnt_s[:, 0:1] += jnp.sum(ohT, axis=1, keepdims=True)

    @pl.when(i == NB - 1)
    def _fin():
        pooled = pool_s[...] / jnp.maximum(cnt_s[:, 0:1], 1.0)
        o2 = jnp.dot(pooled, w2_ref[...], precision=_HIGH) + b2_ref[...]
        z = jax.nn.relu(jnp.dot(o2, fw1_ref[...], precision=_HIGH) + fb1_ref[...])
        res_ref[...] = jnp.dot(z, fw2_ref[...], precision=_HIGH) + fb2_ref[...]


def _tc2(hmid, stats, batch3d, bn_g, bn_b, W2, b2, fcW1, fcb1, fcW2, fcb2):
    return pl.pallas_call(
        _tc2_body,
        grid=(NB,),
        in_specs=[
            pl.BlockSpec((BN, MID), lambda i: (i, 0)),
            pl.BlockSpec((2, MID), lambda i: (0, 0)),
            pl.BlockSpec((1, 1, BN), lambda i: (i, 0, 0)),
            pl.BlockSpec((1, MID), lambda i: (0, 0)),
            pl.BlockSpec((1, MID), lambda i: (0, 0)),
            pl.BlockSpec((MID, D), lambda i: (0, 0)),
            pl.BlockSpec((1, D), lambda i: (0, 0)),
            pl.BlockSpec((D, HID), lambda i: (0, 0)),
            pl.BlockSpec((1, HID), lambda i: (0, 0)),
            pl.BlockSpec((HID, NCLS), lambda i: (0, 0)),
            pl.BlockSpec((1, NCLS), lambda i: (0, 0)),
        ],
        out_specs=pl.BlockSpec((G, NCLS), lambda i: (0, 0)),
        out_shape=jax.ShapeDtypeStruct((G, NCLS), jnp.float32),
        scratch_shapes=[
            pltpu.VMEM((G, MID), jnp.float32),
            pltpu.VMEM((G, 128), jnp.float32),
        ],
        interpret=INTERPRET,
    )(hmid, stats, batch3d, bn_g, bn_b, W2, b2, fcW1, fcb1, fcW2, fcb2)


from jax.experimental.pallas import tpu_sc as plsc

CN = 1920
NPASS = NP_ // (2 * CN)
EPT = E // 16
EPTP = 20480
NCH = EPTP // 128
PADSZ = 16384
CBLK = CN * 512
MBLK = CBLK + PADSZ
STRIPE = CBLK // 16
SLAB = CBLK // 8
GSLAB = (NP_ // 16) * 1024
KCHUNKS = [(0, 4096), (4096, 4096), (8192, 4096), (12288, 4096), (16384, 3616)]


def _sc_body(x_hbm, src_hbm, dst_hbm, ea_hbm, c_hbm,
             x_v, qd_v, qk_v, idx_v, val_v, tmp_s, tmp_e, zeros_v,
             blk, sem):
    c = lax.axis_index("c")
    s = lax.axis_index("s")
    ebase = s * EPT

    @pl.loop(0, 120)
    def _zeros(g):
        zeros_v[pl.ds(g * 16, 16)] = jnp.full((16,), 0, jnp.int32)

    pltpu.sync_copy(x_hbm, x_v)
    pltpu.sync_copy(dst_hbm.at[pl.ds(ebase, EPT)], qd_v.at[pl.ds(0, EPT)])

    @pl.loop(0, (EPTP - EPT) // 16)
    def _tail(k):
        qd_v[pl.ds(EPT + k * 16, 16)] = jnp.full((16,), -1, jnp.int32)

    @pl.loop(0, EPTP // 16)
    def _qd(g):
        qd_v[pl.ds(g * 16, 16)] = qd_v[pl.ds(g * 16, 16)] << 6

    for coff, clen in KCHUNKS:
        pltpu.sync_copy(src_hbm.at[pl.ds(ebase + coff, clen)],
                        tmp_s.at[pl.ds(0, clen)])
        pltpu.sync_copy(ea_hbm.at[pl.ds(ebase + coff, clen)],
                        tmp_e.at[pl.ds(0, clen)])

        @pl.loop(0, clen // 16)
        def _key(g):
            s16 = tmp_s[pl.ds(g * 16, 16)]
            xg = plsc.load_gather(x_v, [s16])
            key = (xg << 4) + tmp_e[pl.ds(g * 16, 16)]
            kh = key >> 7
            qk_v[pl.ds(coff + g * 16, 16)] = (
                (kh << 17) - (kh << 13) + ((key & 127) >> 1) + ((key & 1) << 31))

    iota16 = lax.broadcasted_iota(jnp.int32, (16,), 0)

    for p in range(NPASS):
        base = (2 * p + c) * CN
        boff = base * 64

        @pl.loop(0, STRIPE // 1920)
        def _zero(j):
            pltpu.sync_copy(zeros_v, blk.at[pl.ds(s * STRIPE + j * 1920, 1920)])

        plsc.subcore_barrier()

        @pl.loop(0, NCH // 8)
        def _scat(o):
            descs = []
            for gch in range(8):
                ch = o * 8 + gch
                for g in range(8):
                    off = ch * 128 + g * 16
                    qkr = qk_v[pl.ds(off, 16)]
                    t = qd_v[pl.ds(off, 16)] - boff
                    ok = t.astype(jnp.uint32) < jnp.uint32(SLAB)
                    pad = (CBLK + (off & 16368)) + iota16
                    idx_v[ch % 16, pl.ds(g * 16, 16)] = jnp.where(
                        ok, t + (qkr & 0x7FFFFFFF), pad)
                    val_v[ch % 16, pl.ds(g * 16, 16)] = jnp.where(
                        qkr < 0, 65536, 1)
                descs.append(
                    pltpu.async_copy(val_v.at[ch % 16], blk.at[idx_v.at[ch % 16]],
                                     sem, add=True))
            for dsc in descs:
                dsc.wait()

        plsc.subcore_barrier()

        dd = []
        for kh in range(8):
            dd.append(pltpu.async_copy(
                blk.at[pl.ds(kh * SLAB + s * (SLAB // 16), SLAB // 16)],
                c_hbm.at[pl.ds(kh * GSLAB + boff + s * (SLAB // 16),
                               SLAB // 16)],
                sem))
        for dsc in dd:
            dsc.wait()
        plsc.subcore_barrier()


def _sc_histogram(x, src, dst, ea):
    mesh = plsc.VectorSubcoreMesh(core_axis_name="c", subcore_axis_name="s")
    f = pl.kernel(
        _sc_body,
        out_type=jax.ShapeDtypeStruct((NP_ * 512,), jnp.int32),
        mesh=mesh,
        compiler_params=pltpu.CompilerParams(needs_layout_passes=False),
        scratch_types=[
            pltpu.VMEM((N,), jnp.int32),
            pltpu.VMEM((EPTP,), jnp.int32),
            pltpu.VMEM((EPTP,), jnp.int32),
            pltpu.VMEM((16, 128), jnp.int32),
            pltpu.VMEM((16, 128), jnp.int32),
            pltpu.VMEM((4096,), jnp.int32),
            pltpu.VMEM((4096,), jnp.int32),
            pltpu.VMEM((1920,), jnp.int32),
            pltpu.VMEM_SHARED((MBLK,), jnp.int32),
            pltpu.SemaphoreType.DMA,
        ],
    )
    return f(x, src, dst, ea)


def _histogram(x, edge_index, edge_attr):
    src = edge_index[0]
    dst = edge_index[1]
    c32 = _sc_histogram(x, src, dst, edge_attr)
    c = jax.lax.bitcast_convert_type(c32, jnp.int16)
    return c.reshape(8, NP_ // 16, 16, 128)


def kernel(x, edge_index, edge_attr, batch, v_emb, e_emb, W1, b1, bn_g, bn_b,
           W2, b2, fcW1, fcb1, fcW2, fcb2):
    c = _histogram(x, edge_index, edge_attr)

    x2d = jnp.pad(x, (0, NP_ - N)).reshape(NP_, 1)
    batch3d = jnp.pad(batch, (0, NP_ - N), constant_values=G).reshape(NB, 1, BN)

    hmid, stats = _tc1(c, x2d, v_emb, e_emb, W1, b1.reshape(1, MID))
    res = _tc2(hmid, stats, batch3d,
               bn_g.reshape(1, MID), bn_b.reshape(1, MID),
               W2, b2.reshape(1, D), fcW1, fcb1.reshape(1, HID),
               fcW2, fcb2.reshape(1, NCLS))
    return res


# speedup vs baseline: 1.7350x; 1.7350x over previous
import functools

import numpy as np
import jax
import jax.numpy as jnp
from jax import lax
from jax.experimental import pallas as pl
from jax.experimental.pallas import tpu as pltpu

N = 10000
E = 320000
D = 128
MID = 256
HID = 256
NCLS = 32
NV = 64
NE = 16
G = 64
EPS = 1e-7

BN = 960
NP_ = 11520
NB = NP_ // BN

_HIGH = lax.Precision.HIGHEST

INTERPRET = False


def _tc1_body(c_ref, x_ref, v_ref, e_ref, w1_ref, b1_ref,
              hmid_ref, stats_ref, exp_s, expm_s, acc_s, eer_s):
    i = pl.program_id(0)

    @pl.when(i == 0)
    def _init():
        for p in range(2):
            for l in range(8):
                eer_s[p * 8 + l:p * 8 + l + 1, :] = (
                    e_ref[2 * l + p:2 * l + p + 1, :])
        for vi in range(NV):
            kh = (vi * NE) // 128
            off = ((vi * NE) & 127) // 2
            for p in range(2):
                r0 = kh * 128 + p * 64 + off
                m = jax.nn.relu(v_ref[vi:vi + 1, :]
                                + eer_s[p * 8:(p + 1) * 8, :]) + EPS
                ex = jnp.exp(m)
                exp_s[r0:r0 + 8, :] = ex
                expm_s[r0:r0 + 8, :] = ex * m
        acc_s[...] = jnp.zeros_like(acc_s)

    den = jnp.zeros((BN, D), jnp.float32)
    num = jnp.zeros((BN, D), jnp.float32)
    for kh in range(8):
        cw = c_ref[kh].reshape(BN // 2, 128)
        low = (cw & 0xFFFF).astype(jnp.float32)
        high = (cw >> 16).astype(jnp.float32)
        lh = jnp.concatenate(
            [jnp.concatenate([low[:, 0:64], high[:, 0:64]], axis=1),
             jnp.concatenate([low[:, 64:128], high[:, 64:128]], axis=1)],
            axis=0)
        den = den + jnp.dot(lh, exp_s[kh * 128:(kh + 1) * 128, :])
        num = num + jnp.dot(lh, expm_s[kh * 128:(kh + 1) * 128, :])
    aggr = num / (den + 1e-16)

    xb = x_ref[...]
    oh = (xb == lax.broadcasted_iota(jnp.int32, (1, NV), 1)).astype(jnp.float32)
    h = jnp.dot(oh, v_ref[...], precision=_HIGH)

    out = aggr + h
    hmid = jnp.dot(out, w1_ref[...], precision=_HIGH) + b1_ref[...]
    hmid_ref[...] = hmid

    rloc = lax.broadcasted_iota(jnp.int32, (BN, 1), 0)
    rid = i * BN + jnp.where(rloc < BN // 2, 2 * rloc, 2 * rloc - (BN - 1))
    hm = jnp.where(rid < N, hmid, 0.0)
    acc_s[0:1, :] += jnp.sum(hm, axis=0, keepdims=True)
    acc_s[1:2, :] += jnp.sum(hm * hm, axis=0, keepdims=True)

    @pl.when(i == NB - 1)
    def _fin():
        stats_ref[...] = acc_s[...]


def _tc1(c, x2d, v_emb, e_emb, W1, b1):
    return pl.pallas_call(
        _tc1_body,
        grid=(NB,),
        in_specs=[
            pl.BlockSpec((8, BN // 16, 8, 128), lambda i: (0, i, 0, 0)),
            pl.BlockSpec((BN, 1), lambda i: (i, 0)),
            pl.BlockSpec((NV, D), lambda i: (0, 0)),
            pl.BlockSpec((NE, D), lambda i: (0, 0)),
            pl.BlockSpec((D, MID), lambda i: (0, 0)),
            pl.BlockSpec((1, MID), lambda i: (0, 0)),
        ],
        out_specs=[
            pl.BlockSpec((BN, MID), lambda i: (i, 0)),
            pl.BlockSpec((2, MID), lambda i: (0, 0)),
        ],
        out_shape=[
            jax.ShapeDtypeStruct((NP_, MID), jnp.float32),
            jax.ShapeDtypeStruct((2, MID), jnp.float32),
        ],
        scratch_shapes=[
            pltpu.VMEM((NV * NE, D), jnp.float32),
            pltpu.VMEM((NV * NE, D), jnp.float32),
            pltpu.VMEM((2, MID), jnp.float32),
            pltpu.VMEM((NE, D), jnp.float32),
        ],
        interpret=INTERPRET,
    )(c, x2d, v_emb, e_emb, W1, b1)


def _tc2_body(hmid_ref, stats_ref, batch_ref, bng_ref, bnb_ref,
              w2_ref, b2_ref, fw1_ref, fb1_ref, fw2_ref, fb2_ref,
              res_ref, pool_s, cnt_s):
    i = pl.program_id(0)

    @pl.when(i == 0)
    def _init():
        pool_s[...] = jnp.zeros_like(pool_s)
        cnt_s[...] = jnp.zeros_like(cnt_s)

    st = stats_ref[...]
    mu = st[0:1, :] / N
    var = st[1:2, :] / N - mu * mu
    inv = lax.rsqrt(var + 1e-5)

    hb = hmid_ref[...]
    relu_h = jax.nn.relu((hb - mu) * inv * bng_ref[...] + bnb_ref[...])

    bb = batch_ref[0]
    ohT = (lax.broadcasted_iota(jnp.int32, (G, 1), 0) == bb).astype(jnp.float32)
    pool_s[...] += jnp.dot(ohT, relu_h, precision=_HIGH)
    cnt_s[:, 0:1] += jnp.sum(ohT, axis=1, keepdims=True)

    @pl.when(i == NB - 1)
    def _fin():
        pooled = pool_s[...] / jnp.maximum(cnt_s[:, 0:1], 1.0)
        o2 = jnp.dot(pooled, w2_ref[...], precision=_HIGH) + b2_ref[...]
        z = jax.nn.relu(jnp.dot(o2, fw1_ref[...], precision=_HIGH) + fb1_ref[...])
        res_ref[...] = jnp.dot(z, fw2_ref[...], precision=_HIGH) + fb2_ref[...]


def _tc2(hmid, stats, batch3d, bn_g, bn_b, W2, b2, fcW1, fcb1, fcW2, fcb2):
    return pl.pallas_call(
        _tc2_body,
        grid=(NB,),
        in_specs=[
            pl.BlockSpec((BN, MID), lambda i: (i, 0)),
            pl.BlockSpec((2, MID), lambda i: (0, 0)),
            pl.BlockSpec((1, 1, BN), lambda i: (i, 0, 0)),
            pl.BlockSpec((1, MID), lambda i: (0, 0)),
            pl.BlockSpec((1, MID), lambda i: (0, 0)),
            pl.BlockSpec((MID, D), lambda i: (0, 0)),
            pl.BlockSpec((1, D), lambda i: (0, 0)),
            pl.BlockSpec((D, HID), lambda i: (0, 0)),
            pl.BlockSpec((1, HID), lambda i: (0, 0)),
            pl.BlockSpec((HID, NCLS), lambda i: (0, 0)),
            pl.BlockSpec((1, NCLS), lambda i: (0, 0)),
        ],
        out_specs=pl.BlockSpec((G, NCLS), lambda i: (0, 0)),
        out_shape=jax.ShapeDtypeStruct((G, NCLS), jnp.float32),
        scratch_shapes=[
            pltpu.VMEM((G, MID), jnp.float32),
            pltpu.VMEM((G, 128), jnp.float32),
        ],
        interpret=INTERPRET,
    )(hmid, stats, batch3d, bn_g, bn_b, W2, b2, fcW1, fcb1, fcW2, fcb2)


from jax.experimental.pallas import tpu_sc as plsc

CN = 1920
NPASS = NP_ // (2 * CN)
EPT = E // 16
EPTP = 20480
NCH = EPTP // 128
PADSZ = 16384
CBLK = CN * 512
MBLK = CBLK + PADSZ
STRIPE = CBLK // 16
SLAB = CBLK // 8
GSLAB = (NP_ // 16) * 1024
KCHUNKS = [(0, 4096), (4096, 4096), (8192, 4096), (12288, 4096), (16384, 3616)]


def _sc_body(x_hbm, src_hbm, dst_hbm, ea_hbm, c_hbm,
             x_v, qd_v, qk_v, idx_v, val_v, tmp_s, tmp_e, zeros_v,
             blk, sem):
    c = lax.axis_index("c")
    s = lax.axis_index("s")
    ebase = s * EPT

    @pl.loop(0, 120)
    def _zeros(g):
        zeros_v[pl.ds(g * 16, 16)] = jnp.full((16,), 0, jnp.int32)

    pltpu.sync_copy(x_hbm, x_v)
    pltpu.sync_copy(dst_hbm.at[pl.ds(ebase, EPT)], qd_v.at[pl.ds(0, EPT)])

    @pl.loop(0, (EPTP - EPT) // 16)
    def _tail(k):
        qd_v[pl.ds(EPT + k * 16, 16)] = jnp.full((16,), -1, jnp.int32)

    @pl.loop(0, EPTP // 16)
    def _qd(g):
        qd_v[pl.ds(g * 16, 16)] = qd_v[pl.ds(g * 16, 16)] << 6

    for coff, clen in KCHUNKS:
        pltpu.sync_copy(src_hbm.at[pl.ds(ebase + coff, clen)],
                        tmp_s.at[pl.ds(0, clen)])
        pltpu.sync_copy(ea_hbm.at[pl.ds(ebase + coff, clen)],
                        tmp_e.at[pl.ds(0, clen)])

        @pl.loop(0, clen // 16)
        def _key(g):
            s16 = tmp_s[pl.ds(g * 16, 16)]
            xg = plsc.load_gather(x_v, [s16])
            key = (xg << 4) + tmp_e[pl.ds(g * 16, 16)]
            kh = key >> 7
            qk_v[pl.ds(coff + g * 16, 16)] = (
                (kh << 17) - (kh << 13) + ((key & 127) >> 1) + ((key & 1) << 31))

    iota16 = lax.broadcasted_iota(jnp.int32, (16,), 0)

    for p in range(NPASS):
        base = (2 * p + c) * CN
        boff = base * 64

        @pl.loop(0, STRIPE // 1920)
        def _zero(j):
            pltpu.sync_copy(zeros_v, blk.at[pl.ds(s * STRIPE + j * 1920, 1920)])

        plsc.subcore_barrier()

        @pl.loop(0, NCH // 8)
        def _scat(o):
            descs = []
            for gch in range(8):
                ch = o * 8 + gch
                for g in range(8):
                    off = ch * 128 + g * 16
                    qkr = qk_v[pl.ds(off, 16)]
                    t = qd_v[pl.ds(off, 16)] - boff
                    ok = t.astype(jnp.uint32) < jnp.uint32(SLAB)
                    pad = (CBLK + (off & 16368)) + iota16
                    idx_v[ch % 16, pl.ds(g * 16, 16)] = jnp.where(
                        ok, t + (qkr & 0x7FFFFFFF), pad)
                    val_v[ch % 16, pl.ds(g * 16, 16)] = jnp.where(
                        qkr < 0, 65536, 1)
                descs.append(
                    pltpu.async_copy(val_v.at[ch % 16], blk.at[idx_v.at[ch % 16]],
                                     sem, add=True))
            for dsc in descs:
                dsc.wait()

        plsc.subcore_barrier()

        dd = []
        for kh in range(8):
            dd.append(pltpu.async_copy(
                blk.at[pl.ds(kh * SLAB + s * (SLAB // 16), SLAB // 16)],
                c_hbm.at[pl.ds(kh * GSLAB + boff + s * (SLAB // 16),
                               SLAB // 16)],
                sem))
        for dsc in dd:
            dsc.wait()
        plsc.subcore_barrier()


def _sc_histogram(x, src, dst, ea):
    mesh = plsc.VectorSubcoreMesh(core_axis_name="c", subcore_axis_name="s")
    f = pl.kernel(
        _sc_body,
        out_type=jax.ShapeDtypeStruct((NP_ * 512,), jnp.int32),
        mesh=mesh,
        compiler_params=pltpu.CompilerParams(needs_layout_passes=False),
        scratch_types=[
            pltpu.VMEM((N,), jnp.int32),
            pltpu.VMEM((EPTP,), jnp.int32),
            pltpu.VMEM((EPTP,), jnp.int32),
            pltpu.VMEM((16, 128), jnp.int32),
            pltpu.VMEM((16, 128), jnp.int32),
            pltpu.VMEM((4096,), jnp.int32),
            pltpu.VMEM((4096,), jnp.int32),
            pltpu.VMEM((1920,), jnp.int32),
            pltpu.VMEM_SHARED((MBLK,), jnp.int32),
            pltpu.SemaphoreType.DMA,
        ],
    )
    return f(x, src, dst, ea)


def _histogram(x, edge_index, edge_attr):
    src = edge_index[0]
    dst = edge_index[1]
    c32 = _sc_histogram(x, src, dst, edge_attr)
    return c32.reshape(8, NP_ // 16, 8, 128)


_RL = np.arange(NP_)
_RLOC = _RL % BN
_PERM = (_RL // BN) * BN + np.where(
    _RLOC < BN // 2, 2 * _RLOC, 2 * _RLOC - (BN - 1))


def kernel(x, edge_index, edge_attr, batch, v_emb, e_emb, W1, b1, bn_g, bn_b,
           W2, b2, fcW1, fcb1, fcW2, fcb2):
    c = _histogram(x, edge_index, edge_attr)

    perm = jnp.asarray(_PERM, jnp.int32)
    x2d = jnp.pad(x, (0, NP_ - N))[perm].reshape(NP_, 1)
    batch3d = jnp.pad(batch, (0, NP_ - N),
                      constant_values=G)[perm].reshape(NB, 1, BN)

    hmid, stats = _tc1(c, x2d, v_emb, e_emb, W1, b1.reshape(1, MID))
    res = _tc2(hmid, stats, batch3d,
               bn_g.reshape(1, MID), bn_b.reshape(1, MID),
               W2, b2.reshape(1, D), fcW1, fcb1.reshape(1, HID),
               fcW2, fcb2.reshape(1, NCLS))
    return res
